# baseline (speedup 1.0000x reference)
import jax
import jax.numpy as jnp
from jax import lax
from jax.experimental import pallas as pl
from jax.experimental.pallas import tpu as pltpu
from jax.experimental.pallas import tpu_sc as plsc

_NUM_ENTITIES = 100000
_NUM_RELATIONS = 1000
_DIM = 128
_BATCH = 4096

_info = plsc.get_sparse_core_info()
_NC, _NS = _info.num_cores, _info.num_subcores
_NW = _NC * _NS
_BPW = _BATCH // _NW
_NCHUNK = 4
_CROWS = _BPW // _NCHUNK

_mesh = plsc.VectorSubcoreMesh(core_axis_name="c", subcore_axis_name="s")


@jax.jit
def _lookup_concat(e_ids, r_ids, entity_embeddings, relation_embeddings):
    @pl.kernel(
        out_type=jax.ShapeDtypeStruct((_BATCH, 2 * _DIM), jnp.float32),
        mesh=_mesh,
        scratch_types=[
            pltpu.VMEM((_BPW,), jnp.int32),
            pltpu.VMEM((_BPW,), jnp.int32),
            pltpu.VMEM((_BPW, _DIM), jnp.float32),
            pltpu.VMEM((_BPW, _DIM), jnp.float32),
            pltpu.SemaphoreType.DMA,
            pltpu.SemaphoreType.DMA,
            [pltpu.SemaphoreType.DMA] * _NCHUNK,
            [pltpu.SemaphoreType.DMA] * _NCHUNK,
            pltpu.SemaphoreType.DMA,
        ],
    )
    def k(e_hbm, r_hbm, ent_hbm, rel_hbm, out_hbm,
          eidx_v, ridx_v, erows_v, rrows_v, sem_ei, sem_ri,
          sems_e, sems_r, sem_w):
        wid = lax.axis_index("s") * _NC + lax.axis_index("c")
        base = wid * _BPW
        cp_ei = pltpu.async_copy(e_hbm.at[pl.ds(base, _BPW)], eidx_v, sem_ei)
        cp_ri = pltpu.async_copy(r_hbm.at[pl.ds(base, _BPW)], ridx_v, sem_ri)
        cp_ei.wait()
        cp_ri.wait()
        ge, gr = [], []
        for i in range(_NCHUNK):
            sl = pl.ds(i * _CROWS, _CROWS)
            ge.append(pltpu.async_copy(
                ent_hbm.at[eidx_v.at[sl]], erows_v.at[sl], sems_e[i]))
            gr.append(pltpu.async_copy(
                rel_hbm.at[ridx_v.at[sl]], rrows_v.at[sl], sems_r[i]))
        wr = []
        for i in range(_NCHUNK):
            sl = pl.ds(i * _CROWS, _CROWS)
            osl = pl.ds(base + i * _CROWS, _CROWS)
            ge[i].wait()
            wr.append(pltpu.async_copy(
                erows_v.at[sl], out_hbm.at[osl, pl.ds(0, _DIM)], sem_w))
            gr[i].wait()
            wr.append(pltpu.async_copy(
                rrows_v.at[sl], out_hbm.at[osl, pl.ds(_DIM, _DIM)], sem_w))
        for cp in wr:
            cp.wait()

    return k(e_ids, r_ids, entity_embeddings, relation_embeddings)


def kernel(e_ids, r_ids, entity_embeddings, relation_embeddings):
    return _lookup_concat(e_ids, r_ids, entity_embeddings, relation_embeddings)


# speedup vs baseline: 1.0146x; 1.0146x over previous
import jax
import jax.numpy as jnp
from jax import lax
from jax.experimental import pallas as pl
from jax.experimental.pallas import tpu as pltpu
from jax.experimental.pallas import tpu_sc as plsc

_NUM_ENTITIES = 100000
_NUM_RELATIONS = 1000
_DIM = 128
_BATCH = 4096

_info = plsc.get_sparse_core_info()
_NC, _NS = _info.num_cores, _info.num_subcores
_NW = _NC * _NS
_BPW = _BATCH // _NW
_NCHUNK = 4
_CROWS = _BPW // _NCHUNK

_mesh = plsc.VectorSubcoreMesh(core_axis_name="c", subcore_axis_name="s")


@jax.jit
def _lookup_concat(e_ids, r_ids, entity_embeddings, relation_embeddings):
    @pl.kernel(
        out_type=jax.ShapeDtypeStruct((_BATCH, 2 * _DIM), jnp.float32),
        mesh=_mesh,
        scratch_types=[
            pltpu.VMEM((_BPW,), jnp.int32),
            pltpu.VMEM((_BPW,), jnp.int32),
            pltpu.VMEM((_BPW, 2 * _DIM), jnp.float32),
            pltpu.SemaphoreType.DMA,
            pltpu.SemaphoreType.DMA,
            pltpu.SemaphoreType.DMA,
        ],
    )
    def k(e_hbm, r_hbm, ent_hbm, rel_hbm, out_hbm,
          eidx_v, ridx_v, cat_v, sem_e, sem_r, sem_w):
        wid = lax.axis_index("s") * _NC + lax.axis_index("c")
        base = wid * _BPW
        cp_ei = pltpu.async_copy(e_hbm.at[pl.ds(base, _BPW)], eidx_v, sem_e)
        cp_ri = pltpu.async_copy(r_hbm.at[pl.ds(base, _BPW)], ridx_v, sem_r)
        cp_ei.wait()
        cp_e = pltpu.async_copy(
            ent_hbm.at[eidx_v], cat_v.at[:, pl.ds(0, _DIM)], sem_e)
        cp_ri.wait()
        cp_r = pltpu.async_copy(
            rel_hbm.at[ridx_v], cat_v.at[:, pl.ds(_DIM, _DIM)], sem_r)
        cp_e.wait()
        cp_r.wait()
        pltpu.async_copy(
            cat_v, out_hbm.at[pl.ds(base, _BPW)], sem_w).wait()

    return k(e_ids, r_ids, entity_embeddings, relation_embeddings)


def kernel(e_ids, r_ids, entity_embeddings, relation_embeddings):
    return _lookup_concat(e_ids, r_ids, entity_embeddings, relation_embeddings)
